# baseline (speedup 1.0000x reference)
import functools

import jax
import jax.numpy as jnp
from jax import lax
from jax.experimental import pallas as pl
from jax.experimental.pallas import tpu as pltpu
from jax.experimental.pallas import tpu_sc as plsc

N_CAT = 26
N_NUM = 13
B = 16384
VOCAB = 100001
D = 32

NC = 2
NS = 16
NW = NC * NS
BPW = B // NW
GCH = 128
NG = BPW // GCH


def _body(tab, cat, num, wx, bx, out, idxr, rows, numbuf, numr, wxr, bxr, sem):
    wid = lax.axis_index("s") * NC + lax.axis_index("c")
    base = wid * BPW

    pltpu.sync_copy(cat.at[:, pl.ds(wid * NG, NG), :], idxr)
    pltpu.sync_copy(num.at[:, pl.ds(base, BPW)], numr)
    pltpu.sync_copy(wx, wxr)
    pltpu.sync_copy(bx, bxr)

    for f in range(N_CAT):
        for k in range(NG):
            def obody(j, c, f=f, k=k):
                sl = pl.ds(j * 16, 16)
                idxr[f, k, sl] = idxr[f, k, sl] + (f * VOCAB)
                return c
            lax.fori_loop(0, GCH // 16, obody, 0)

    for f in range(N_CAT):
        descs = [
            pltpu.async_copy(tab.at[idxr.at[f, k]],
                             rows.at[pl.ds(k * GCH, GCH)], sem)
            for k in range(NG)
        ]
        for d_ in descs:
            d_.wait()
        pltpu.sync_copy(rows, out.at[pl.ds(base, BPW), pl.ds(f * D, D)])

    iota16 = lax.iota(jnp.int32, 16)
    for n in range(N_NUM):
        def dbody(d, c, n=n):
            wv = wxr[n * D + d, :]
            bv = bxr[n * D + d, :]
            didx = jnp.full((16,), 0, jnp.int32) + d
            def bbody(bb, c2):
                v = numr[n, pl.ds(bb * 16, 16)] * wv + bv
                plsc.store_scatter(numbuf, [bb * 16 + iota16, didx], v)
                return c2
            return lax.fori_loop(0, BPW // 16, bbody, c)
        lax.fori_loop(0, D, dbody, 0)
        pltpu.sync_copy(numbuf,
                        out.at[pl.ds(base, BPW),
                               pl.ds((N_CAT + n) * D, D)])


@jax.jit
def _run(tabflat, cat3, num, wx, bx):
    mesh = plsc.VectorSubcoreMesh(core_axis_name="c", subcore_axis_name="s",
                                  num_cores=NC, num_subcores=NS)
    return pl.kernel(
        _body,
        out_type=jax.ShapeDtypeStruct((B, (N_CAT + N_NUM) * D), jnp.float32),
        mesh=mesh,
        compiler_params=pltpu.CompilerParams(use_tc_tiling_on_sc=False,
                                             needs_layout_passes=False),
        scratch_types=[
            pltpu.VMEM((N_CAT, NG, GCH), jnp.int32),
            pltpu.VMEM((BPW, D), jnp.float32),
            pltpu.VMEM((BPW, D), jnp.float32),
            pltpu.VMEM((N_NUM, BPW), jnp.float32),
            pltpu.VMEM((N_NUM * D, 16), jnp.float32),
            pltpu.VMEM((N_NUM * D, 16), jnp.float32),
            pltpu.SemaphoreType.DMA,
        ],
    )(tabflat, cat3, num, wx, bx)


def kernel(cat_features, num_features, tables, W, b):
    tabflat = tables.reshape(N_CAT * VOCAB, D)
    cat3 = cat_features.reshape(N_CAT, B // GCH, GCH)
    wx = jnp.broadcast_to(W.reshape(N_NUM * D)[:, None], (N_NUM * D, 16))
    bx = jnp.broadcast_to(b.reshape(N_NUM * D)[:, None], (N_NUM * D, 16))
    return _run(tabflat, cat3, num_features, wx, bx)


# speedup vs baseline: 23.7007x; 23.7007x over previous
import jax
import jax.numpy as jnp
from jax import lax
from jax.experimental import pallas as pl
from jax.experimental.pallas import tpu as pltpu
from jax.experimental.pallas import tpu_sc as plsc

N_CAT = 26
N_NUM = 13
B = 16384
VOCAB = 100001
D = 32

NC = 2
NS = 16
NW = NC * NS
HALF = B // 2


def _body(tabT, cat, num, wx, bx, out, plane, idxf, ob, wsc, bsc, sem):
    wid = lax.axis_index("s") * NC + lax.axis_index("c")
    d = wid

    for f in range(N_CAT):
        pltpu.sync_copy(cat.at[pl.ds(f, 1), :], idxf)
        pltpu.sync_copy(tabT.at[f, pl.ds(d, 1)], plane)
        for half in range(2):
            def gbody(i, c, half=half):
                iv = idxf[0, pl.ds(half * HALF + i * 16, 16)]
                ob[0, pl.ds(i * 16, 16)] = plsc.load_gather(plane.at[0], [iv])
                return c
            lax.fori_loop(0, HALF // 16, gbody, 0)
            pltpu.sync_copy(ob, out.at[pl.ds(f * D + d, 1),
                                       pl.ds(half * HALF, HALF)])

    for n in range(N_NUM):
        p = n * D + d
        pltpu.sync_copy(wx.at[pl.ds(p, 1)], wsc)
        pltpu.sync_copy(bx.at[pl.ds(p, 1)], bsc)
        wv = wsc[0, :]
        bv = bsc[0, :]
        for half in range(2):
            pltpu.sync_copy(num.at[pl.ds(n, 1), pl.ds(half * HALF, HALF)], ob)
            def nbody(i, c):
                sl = pl.ds(i * 16, 16)
                ob[0, sl] = ob[0, sl] * wv + bv
                return c
            lax.fori_loop(0, HALF // 16, nbody, 0)
            pltpu.sync_copy(ob, out.at[pl.ds((N_CAT + n) * D + d, 1),
                                       pl.ds(half * HALF, HALF)])


@jax.jit
def _run(tabT, cat, num, wx, bx):
    mesh = plsc.VectorSubcoreMesh(core_axis_name="c", subcore_axis_name="s",
                                  num_cores=NC, num_subcores=NS)
    return pl.kernel(
        _body,
        out_type=jax.ShapeDtypeStruct(((N_CAT + N_NUM) * D, B), jnp.float32),
        mesh=mesh,
        compiler_params=pltpu.CompilerParams(needs_layout_passes=False),
        scratch_types=[
            pltpu.VMEM((1, VOCAB), jnp.float32),
            pltpu.VMEM((1, B), jnp.int32),
            pltpu.VMEM((1, HALF), jnp.float32),
            pltpu.VMEM((1, 16), jnp.float32),
            pltpu.VMEM((1, 16), jnp.float32),
            pltpu.SemaphoreType.DMA,
        ],
    )(tabT, cat, num, wx, bx)


def kernel(cat_features, num_features, tables, W, b):
    tabT = jnp.transpose(tables, (0, 2, 1))
    wx = jnp.broadcast_to(W.reshape(N_NUM * D)[:, None], (N_NUM * D, 16))
    bx = jnp.broadcast_to(b.reshape(N_NUM * D)[:, None], (N_NUM * D, 16))
    out_dm = _run(tabT, cat_features, num_features, wx, bx)
    return out_dm.T


# speedup vs baseline: 46.6279x; 1.9674x over previous
import jax
import jax.numpy as jnp
from jax import lax
from jax.experimental import pallas as pl
from jax.experimental.pallas import tpu as pltpu
from jax.experimental.pallas import tpu_sc as plsc

N_CAT = 26
N_NUM = 13
B = 16384
VOCAB = 100001
D = 32

NC = 2
NS = 16
NW = NC * NS
Q = 4096
NQ = B // Q


def _body(tabT, cat, num, wx, bx, out,
          plane, idx0, idx1, ob0, ob1, nb0, nb1, wsc, bsc,
          psem, isem, wsem, nsem, nwsem):
    wid = lax.axis_index("s") * NC + lax.axis_index("c")
    d = wid

    idxb = [idx0, idx1]
    obb = [ob0, ob1]
    nbb = [nb0, nb1]
    wdesc = [None, None]
    nwdesc = [None, None]

    def process_num(n):
        p = n * D + d
        pltpu.sync_copy(wx.at[pl.ds(p, 1)], wsc)
        pltpu.sync_copy(bx.at[pl.ds(p, 1)], bsc)
        wv = wsc[0, :]
        bv = bsc[0, :]
        row = (N_CAT + n) * D + d
        for q in range(NQ):
            bidx = q % 2
            nb = nbb[bidx]
            if nwdesc[bidx] is not None:
                nwdesc[bidx].wait()
            pltpu.async_copy(num.at[pl.ds(n, 1), pl.ds(q * Q, Q)],
                             nb, nsem).wait()

            @plsc.parallel_loop(0, Q // 16, unroll=8)
            def _(i, nb=nb, wv=wv, bv=bv):
                sl = pl.ds(i * 16, 16)
                nb[0, sl] = nb[0, sl] * wv + bv

            nwdesc[bidx] = pltpu.async_copy(
                nb, out.at[pl.ds(row, 1), pl.ds(q * Q, Q)], nwsem)

    for f in range(N_CAT):
        pd = pltpu.async_copy(tabT.at[f, pl.ds(d, 1)], plane, psem)
        idesc = [pltpu.async_copy(cat.at[pl.ds(f, 1), pl.ds(q * Q, Q)],
                                  idxb[q], isem)
                 for q in range(2)]
        if f % 2 == 0 and f // 2 < N_NUM:
            process_num(f // 2)
        pd.wait()
        row = f * D + d
        for q in range(NQ):
            bidx = q % 2
            idesc[bidx].wait()
            ob = obb[bidx]
            if wdesc[bidx] is not None:
                wdesc[bidx].wait()

            @plsc.parallel_loop(0, Q // 16, unroll=8)
            def _(i, ob=ob, idxr=idxb[bidx]):
                iv = idxr[0, pl.ds(i * 16, 16)]
                ob[0, pl.ds(i * 16, 16)] = plsc.load_gather(plane.at[0], [iv])

            wdesc[bidx] = pltpu.async_copy(
                ob, out.at[pl.ds(row, 1), pl.ds(q * Q, Q)], wsem)
            if q + 2 < NQ:
                idesc[bidx] = pltpu.async_copy(
                    cat.at[pl.ds(f, 1), pl.ds((q + 2) * Q, Q)],
                    idxb[bidx], isem)

    for dd in wdesc + nwdesc:
        if dd is not None:
            dd.wait()


@jax.jit
def _run(tabT, cat, num, wx, bx):
    mesh = plsc.VectorSubcoreMesh(core_axis_name="c", subcore_axis_name="s",
                                  num_cores=NC, num_subcores=NS)
    return pl.kernel(
        _body,
        out_type=jax.ShapeDtypeStruct(((N_CAT + N_NUM) * D, B), jnp.float32),
        mesh=mesh,
        compiler_params=pltpu.CompilerParams(needs_layout_passes=False),
        scratch_types=[
            pltpu.VMEM((1, VOCAB), jnp.float32),
            pltpu.VMEM((1, Q), jnp.int32),
            pltpu.VMEM((1, Q), jnp.int32),
            pltpu.VMEM((1, Q), jnp.float32),
            pltpu.VMEM((1, Q), jnp.float32),
            pltpu.VMEM((1, Q), jnp.float32),
            pltpu.VMEM((1, Q), jnp.float32),
            pltpu.VMEM((1, 16), jnp.float32),
            pltpu.VMEM((1, 16), jnp.float32),
            pltpu.SemaphoreType.DMA,
            pltpu.SemaphoreType.DMA,
            pltpu.SemaphoreType.DMA,
            pltpu.SemaphoreType.DMA,
            pltpu.SemaphoreType.DMA,
        ],
    )(tabT, cat, num, wx, bx)


def kernel(cat_features, num_features, tables, W, b):
    tabT = jnp.transpose(tables, (0, 2, 1))
    wx = jnp.broadcast_to(W.reshape(N_NUM * D)[:, None], (N_NUM * D, 16))
    bx = jnp.broadcast_to(b.reshape(N_NUM * D)[:, None], (N_NUM * D, 16))
    out_dm = _run(tabT, cat_features, num_features, wx, bx)
    return out_dm.T
